# Initial kernel scaffold; baseline (speedup 1.0000x reference)
import jax
import jax.numpy as jnp
from jax.experimental import pallas as pl


def kernel(means2D, cov2d, color, opacity, depths):
    raise NotImplementedError("write your pallas kernel here")


# speedup vs baseline: 15.5905x; 15.5905x over previous
import jax
import jax.numpy as jnp
from jax.experimental import pallas as pl

H = 64
W = 64
TS = 32
N = 4096
P = H * W
K = 256
NCHUNK = N // K

_F32 = jnp.float32
_HIGH = jax.lax.Precision.HIGHEST


def _blend_kernel(params_ref, cdt_ref, covu_ref, out_ref, radii_ref):
    c00u = covu_ref[0:1, :]
    c01u = covu_ref[1:2, :]
    c10u = covu_ref[2:3, :]
    c11u = covu_ref[3:4, :]
    detu = c00u * c11u - c01u * c10u
    midu = 0.5 * (c00u + c11u)
    su = jnp.sqrt(jnp.maximum(midu * midu - detu, 0.1))
    radii_ref[...] = 3.0 * jnp.ceil(
        jnp.sqrt(jnp.maximum(midu + su, midu - su)))

    p_idx = jax.lax.broadcasted_iota(jnp.int32, (1, P), 1)
    yi = p_idx // W
    xi = p_idx - yi * W
    y = yi.astype(_F32)
    x = xi.astype(_F32)
    h0 = ((yi // TS) * TS).astype(_F32)
    w0 = ((xi // TS) * TS).astype(_F32)

    ri = jax.lax.broadcasted_iota(jnp.int32, (K, K), 0)
    ci = jax.lax.broadcasted_iota(jnp.int32, (K, K), 1)
    tri = (ci < ri).astype(_F32)

    def body(k, carry):
        log_t, acc, cnt = carry
        pblk = params_ref[pl.ds(k * K, K), :]
        m0 = pblk[:, 0:1]
        m1 = pblk[:, 1:2]
        c00 = pblk[:, 2:3]
        c01 = pblk[:, 3:4]
        c10 = pblk[:, 4:5]
        c11 = pblk[:, 5:6]
        op = pblk[:, 6:7]

        det = c00 * c11 - c01 * c10
        inv_det = 1.0 / det
        i00 = c11 * inv_det
        i11 = c00 * inv_det
        ixy = -(c01 + c10) * inv_det

        mid = 0.5 * (c00 + c11)
        s = jnp.sqrt(jnp.maximum(mid * mid - det, 0.1))
        rad = 3.0 * jnp.ceil(jnp.sqrt(jnp.maximum(mid + s, mid - s)))

        rmin0 = jnp.clip(m0 - rad, 0.0, W - 1.0)
        rmax0 = jnp.clip(m0 + rad, 0.0, W - 1.0)
        rmin1 = jnp.clip(m1 - rad, 0.0, H - 1.0)
        rmax1 = jnp.clip(m1 + rad, 0.0, H - 1.0)
        mask = ((rmax0 >= w0) & (rmin0 <= w0 + (TS - 1.0))
                & (rmax1 >= h0) & (rmin1 <= h0 + (TS - 1.0)))
        mask_f = mask.astype(_F32)

        u = y - m0
        v = x - m1
        mahal = (u * u) * i00 + (u * v) * ixy + (v * v) * i11
        aval = op * jnp.exp(-0.5 * mahal) * mask_f
        lg = jnp.log((1.0 - aval) + 1e-10)

        csum = jax.lax.dot(tri, lg, precision=_HIGH)
        tvals = jnp.exp(csum + log_t)
        bvals = tvals * aval

        cdt = cdt_ref[:, pl.ds(k * K, K)]
        acc = acc + jax.lax.dot(cdt, bvals, precision=_HIGH)
        log_t = log_t + jnp.sum(lg, axis=0, keepdims=True)
        cnt = cnt + jnp.sum(mask_f, axis=0, keepdims=True)
        return log_t, acc, cnt

    log_t0 = jnp.zeros((1, P), _F32)
    acc0 = jnp.zeros((8, P), _F32)
    cnt0 = jnp.zeros((1, P), _F32)
    log_t, acc, cnt = jax.lax.fori_loop(0, NCHUNK, body,
                                        (log_t0, acc0, cnt0))

    default = (jax.lax.broadcasted_iota(jnp.int32, (8, P), 0) < 3).astype(_F32)
    out_ref[...] = jnp.where(cnt > 0.0, acc, default)


def kernel(means2D, cov2d, color, opacity, depths):
    order = jnp.argsort(depths)
    ms = means2D[order]
    cs = cov2d[order].reshape(N, 4)
    ops = opacity[order].reshape(N, 1)
    cols = color[order]
    ds = depths[order].reshape(N, 1)

    params = jnp.concatenate(
        [ms, cs, ops, jnp.zeros((N, 1), _F32)], axis=1)
    cdt = jnp.concatenate(
        [cols, ds, jnp.ones((N, 1), _F32), jnp.zeros((N, 3), _F32)],
        axis=1).T
    covu = jnp.concatenate(
        [cov2d.reshape(N, 4).T, jnp.zeros((4, N), _F32)], axis=0)

    out, radii = pl.pallas_call(
        _blend_kernel,
        out_shape=[
            jax.ShapeDtypeStruct((8, P), _F32),
            jax.ShapeDtypeStruct((1, N), _F32),
        ],
    )(params, cdt, covu)

    render_color = out[0:3].T.reshape(H, W, 3)
    render_depth = out[3].reshape(H, W, 1)
    render_alpha = out[4].reshape(H, W, 1)
    return render_color, render_depth, render_alpha, radii.reshape(N)


# speedup vs baseline: 18.2833x; 1.1727x over previous
import jax
import jax.numpy as jnp
from jax.experimental import pallas as pl

H = 64
W = 64
TS = 32
N = 4096
P = H * W
K = 256
KA = K + 8
NCHUNK = N // K

_F32 = jnp.float32
_BF16 = jnp.bfloat16
_HIGH = jax.lax.Precision.HIGHEST
_LOG2E = 1.4426950408889634


def _dot_f32(a, b):
    return jax.lax.dot(a, b, preferred_element_type=_F32)


def _blend_kernel(params_ref, cdt_ref, covu_ref, out_ref, radii_ref):
    c00u = covu_ref[0:1, :]
    c01u = covu_ref[1:2, :]
    c10u = covu_ref[2:3, :]
    c11u = covu_ref[3:4, :]
    detu = c00u * c11u - c01u * c10u
    midu = 0.5 * (c00u + c11u)
    su = jnp.sqrt(jnp.maximum(midu * midu - detu, 0.1))
    radii_ref[...] = 3.0 * jnp.ceil(
        jnp.sqrt(jnp.maximum(midu + su, midu - su)))

    p_idx = jax.lax.broadcasted_iota(jnp.int32, (1, P), 1)
    yi = p_idx // W
    xi = p_idx - yi * W
    y = yi.astype(_F32)
    x = xi.astype(_F32)
    is_top = yi < TS
    is_left = xi < TS

    ri = jax.lax.broadcasted_iota(jnp.int32, (KA, K), 0)
    ci = jax.lax.broadcasted_iota(jnp.int32, (KA, K), 1)
    tri = (ci < ri).astype(_BF16)

    def body(k, carry):
        log_t, acc, ctl, ctr, cbl, cbr = carry
        pblk = params_ref[pl.ds(k * K, K), :]
        m0 = pblk[:, 0:1]
        m1 = pblk[:, 1:2]
        c00 = pblk[:, 2:3]
        c01 = pblk[:, 3:4]
        c10 = pblk[:, 4:5]
        c11 = pblk[:, 5:6]
        op = pblk[:, 6:7]

        det = c00 * c11 - c01 * c10
        inv_det = 1.0 / det
        pa = (-0.5 * _LOG2E) * (c11 * inv_det)
        pb = (0.5 * _LOG2E) * ((c01 + c10) * inv_det)
        pc = (-0.5 * _LOG2E) * (c00 * inv_det)
        pd = jnp.log2(op)

        mid = 0.5 * (c00 + c11)
        s = jnp.sqrt(jnp.maximum(mid * mid - det, 0.1))
        rad = 3.0 * jnp.ceil(jnp.sqrt(jnp.maximum(mid + s, mid - s)))

        rmin0 = jnp.clip(m0 - rad, 0.0, W - 1.0)
        rmax0 = jnp.clip(m0 + rad, 0.0, W - 1.0)
        rmin1 = jnp.clip(m1 - rad, 0.0, H - 1.0)
        rmax1 = jnp.clip(m1 + rad, 0.0, H - 1.0)
        mwl = (rmax0 >= 0.0) & (rmin0 <= TS - 1.0)
        mwr = (rmax0 >= float(TS)) & (rmin0 <= W - 1.0)
        mht = (rmax1 >= 0.0) & (rmin1 <= TS - 1.0)
        mhb = (rmax1 >= float(TS)) & (rmin1 <= H - 1.0)
        mtl = (mht & mwl).astype(_F32)
        mtr = (mht & mwr).astype(_F32)
        mbl = (mhb & mwl).astype(_F32)
        mbr = (mhb & mwr).astype(_F32)
        mask_f = jnp.where(is_top,
                           jnp.where(is_left, mtl, mtr),
                           jnp.where(is_left, mbl, mbr))

        u = y - m0
        v = x - m1
        e2 = u * (pa * u + pb * v) + pc * (v * v) + pd
        aval = jnp.exp2(e2) * mask_f
        lg = jnp.log2((1.0 - aval) + 1e-10)

        lg_hi = lg.astype(_BF16)
        lg_lo = (lg - lg_hi.astype(_F32)).astype(_BF16)
        cs = _dot_f32(tri, lg_hi) + _dot_f32(tri, lg_lo)
        tvals = jnp.exp2(cs[0:K, :] + log_t)
        bvals = tvals * aval

        cdt = cdt_ref[:, pl.ds(k * K, K)]
        acc = acc + jax.lax.dot(cdt, bvals, precision=_HIGH)
        log_t = log_t + cs[K:K + 1, :]
        ctl = ctl + jnp.sum(mtl)
        ctr = ctr + jnp.sum(mtr)
        cbl = cbl + jnp.sum(mbl)
        cbr = cbr + jnp.sum(mbr)
        return log_t, acc, ctl, ctr, cbl, cbr

    log_t0 = jnp.zeros((1, P), _F32)
    acc0 = jnp.zeros((8, P), _F32)
    zero = jnp.float32(0.0)
    log_t, acc, ctl, ctr, cbl, cbr = jax.lax.fori_loop(
        0, NCHUNK, body, (log_t0, acc0, zero, zero, zero, zero))

    cnt = jnp.where(is_top,
                    jnp.where(is_left, ctl, ctr),
                    jnp.where(is_left, cbl, cbr))
    default = (jax.lax.broadcasted_iota(jnp.int32, (8, P), 0) < 3).astype(_F32)
    out_ref[...] = jnp.where(cnt > 0.0, acc, default)


def kernel(means2D, cov2d, color, opacity, depths):
    order = jnp.argsort(depths)
    ms = means2D[order]
    cs = cov2d[order].reshape(N, 4)
    ops = opacity[order].reshape(N, 1)
    cols = color[order]
    ds = depths[order].reshape(N, 1)

    params = jnp.concatenate(
        [ms, cs, ops, jnp.zeros((N, 1), _F32)], axis=1)
    cdt = jnp.concatenate(
        [cols, ds, jnp.ones((N, 1), _F32), jnp.zeros((N, 3), _F32)],
        axis=1).T
    covu = jnp.concatenate(
        [cov2d.reshape(N, 4).T, jnp.zeros((4, N), _F32)], axis=0)

    out, radii = pl.pallas_call(
        _blend_kernel,
        out_shape=[
            jax.ShapeDtypeStruct((8, P), _F32),
            jax.ShapeDtypeStruct((1, N), _F32),
        ],
    )(params, cdt, covu)

    render_color = out[0:3].T.reshape(H, W, 3)
    render_depth = out[3].reshape(H, W, 1)
    render_alpha = out[4].reshape(H, W, 1)
    return render_color, render_depth, render_alpha, radii.reshape(N)


# speedup vs baseline: 20.7782x; 1.1365x over previous
import jax
import jax.numpy as jnp
from jax.experimental import pallas as pl

H = 64
W = 64
TS = 32
N = 4096
P = H * W
K = 256
KA = K + 8
NCHUNK = N // K

_F32 = jnp.float32
_BF16 = jnp.bfloat16
_HIGH = jax.lax.Precision.HIGHEST
_LOG2E = 1.4426950408889634


def _dot_f32(a, b):
    return jax.lax.dot(a, b, preferred_element_type=_F32)


def _blend_kernel(params_ref, cdt2_ref, covu_ref, out_ref, radii_ref):
    c00u = covu_ref[0:1, :]
    c01u = covu_ref[1:2, :]
    c10u = covu_ref[2:3, :]
    c11u = covu_ref[3:4, :]
    detu = c00u * c11u - c01u * c10u
    midu = 0.5 * (c00u + c11u)
    su = jnp.sqrt(jnp.maximum(midu * midu - detu, 0.1))
    radii_ref[...] = 3.0 * jnp.ceil(
        jnp.sqrt(jnp.maximum(midu + su, midu - su)))

    p_idx = jax.lax.broadcasted_iota(jnp.int32, (1, P), 1)
    yi = p_idx // W
    xi = p_idx - yi * W
    y = yi.astype(_F32)
    x = xi.astype(_F32)
    is_top = yi < TS
    is_left = xi < TS

    ri = jax.lax.broadcasted_iota(jnp.int32, (KA, K), 0)
    ci = jax.lax.broadcasted_iota(jnp.int32, (KA, K), 1)
    tri = (ci < ri).astype(_BF16)

    def body(k, carry):
        log_t, acc, ctl, ctr, cbl, cbr = carry
        pblk = params_ref[pl.ds(k * K, K), :]
        m0 = pblk[:, 0:1]
        m1 = pblk[:, 1:2]
        c00 = pblk[:, 2:3]
        c01 = pblk[:, 3:4]
        c10 = pblk[:, 4:5]
        c11 = pblk[:, 5:6]
        op = pblk[:, 6:7]

        det = c00 * c11 - c01 * c10
        inv_det = 1.0 / det
        pa = (-0.5 * _LOG2E) * (c11 * inv_det)
        pb = (0.5 * _LOG2E) * ((c01 + c10) * inv_det)
        pc = (-0.5 * _LOG2E) * (c00 * inv_det)
        pd = jnp.log2(op)

        mid = 0.5 * (c00 + c11)
        s = jnp.sqrt(jnp.maximum(mid * mid - det, 0.1))
        rad = 3.0 * jnp.ceil(jnp.sqrt(jnp.maximum(mid + s, mid - s)))

        rmin0 = jnp.clip(m0 - rad, 0.0, W - 1.0)
        rmax0 = jnp.clip(m0 + rad, 0.0, W - 1.0)
        rmin1 = jnp.clip(m1 - rad, 0.0, H - 1.0)
        rmax1 = jnp.clip(m1 + rad, 0.0, H - 1.0)
        mwl = (rmax0 >= 0.0) & (rmin0 <= TS - 1.0)
        mwr = (rmax0 >= float(TS)) & (rmin0 <= W - 1.0)
        mht = (rmax1 >= 0.0) & (rmin1 <= TS - 1.0)
        mhb = (rmax1 >= float(TS)) & (rmin1 <= H - 1.0)
        mtl = (mht & mwl).astype(_F32)
        mtr = (mht & mwr).astype(_F32)
        mbl = (mhb & mwl).astype(_F32)
        mbr = (mhb & mwr).astype(_F32)
        mask_f = jnp.where(is_top,
                           jnp.where(is_left, mtl, mtr),
                           jnp.where(is_left, mbl, mbr))

        u = y - m0
        v = x - m1
        e2 = u * (pa * u + pb * v) + pc * (v * v) + pd
        aval = jnp.exp2(e2) * mask_f
        lg = jnp.log2((1.0 - aval) + 1e-10)

        lg_hi = lg.astype(_BF16)
        lg_lo = (lg - lg_hi.astype(_F32)).astype(_BF16)
        cs = _dot_f32(tri, lg_hi) + _dot_f32(tri, lg_lo)
        tvals = jnp.exp2(cs[0:K, :] + log_t)
        bvals = tvals * aval

        cdt2 = cdt2_ref[:, pl.ds(k * K, K)]
        b_hi = bvals.astype(_BF16)
        b_lo = (bvals - b_hi.astype(_F32)).astype(_BF16)
        a2p = _dot_f32(cdt2, b_hi)
        acc = acc + a2p[0:8, :] + a2p[8:16, :] + _dot_f32(cdt2[0:8, :], b_lo)
        log_t = log_t + cs[K:K + 1, :]
        ctl = ctl + jnp.sum(mtl)
        ctr = ctr + jnp.sum(mtr)
        cbl = cbl + jnp.sum(mbl)
        cbr = cbr + jnp.sum(mbr)
        return log_t, acc, ctl, ctr, cbl, cbr

    log_t0 = jnp.zeros((1, P), _F32)
    acc0 = jnp.zeros((8, P), _F32)
    zero = jnp.float32(0.0)
    log_t, acc, ctl, ctr, cbl, cbr = jax.lax.fori_loop(
        0, NCHUNK, body, (log_t0, acc0, zero, zero, zero, zero))

    cnt = jnp.where(is_top,
                    jnp.where(is_left, ctl, ctr),
                    jnp.where(is_left, cbl, cbr))
    default = (jax.lax.broadcasted_iota(jnp.int32, (8, P), 0) < 3).astype(_F32)
    out_ref[...] = jnp.where(cnt > 0.0, acc, default)


def kernel(means2D, cov2d, color, opacity, depths):
    order = jnp.argsort(depths)
    ms = means2D[order]
    cs = cov2d[order].reshape(N, 4)
    ops = opacity[order].reshape(N, 1)
    cols = color[order]
    ds = depths[order].reshape(N, 1)

    params = jnp.concatenate(
        [ms, cs, ops, jnp.zeros((N, 1), _F32)], axis=1)
    cdt = jnp.concatenate(
        [cols, ds, jnp.ones((N, 1), _F32), jnp.zeros((N, 3), _F32)],
        axis=1).T
    cdt_hi = cdt.astype(_BF16)
    cdt_lo = (cdt - cdt_hi.astype(_F32)).astype(_BF16)
    cdt2 = jnp.concatenate([cdt_hi, cdt_lo], axis=0)
    covu = jnp.concatenate(
        [cov2d.reshape(N, 4).T, jnp.zeros((4, N), _F32)], axis=0)

    out, radii = pl.pallas_call(
        _blend_kernel,
        out_shape=[
            jax.ShapeDtypeStruct((8, P), _F32),
            jax.ShapeDtypeStruct((1, N), _F32),
        ],
    )(params, cdt2, covu)

    render_color = out[0:3].T.reshape(H, W, 3)
    render_depth = out[3].reshape(H, W, 1)
    render_alpha = out[4].reshape(H, W, 1)
    return render_color, render_depth, render_alpha, radii.reshape(N)


# speedup vs baseline: 24.3987x; 1.1742x over previous
import jax
import jax.numpy as jnp
from jax.experimental import pallas as pl

H = 64
W = 64
TS = 32
N = 4096
P = H * W
K = 256
KA = K + 8
NCHUNK = N // K

_F32 = jnp.float32
_BF16 = jnp.bfloat16
_HIGH = jax.lax.Precision.HIGHEST
_LOG2E = 1.4426950408889634


def _dot_f32(a, b):
    return jax.lax.dot(a, b, preferred_element_type=_F32)


def _blend_kernel(params_ref, cdt2_ref, covu_ref, out_ref, radii_ref):
    c00u = covu_ref[0:1, :]
    c01u = covu_ref[1:2, :]
    c10u = covu_ref[2:3, :]
    c11u = covu_ref[3:4, :]
    detu = c00u * c11u - c01u * c10u
    midu = 0.5 * (c00u + c11u)
    su = jnp.sqrt(jnp.maximum(midu * midu - detu, 0.1))
    radii_ref[...] = 3.0 * jnp.ceil(
        jnp.sqrt(jnp.maximum(midu + su, midu - su)))

    p_idx = jax.lax.broadcasted_iota(jnp.int32, (1, P), 1)
    yi = p_idx // W
    xi = p_idx - yi * W
    y = yi.astype(_F32)
    x = xi.astype(_F32)
    is_top = yi < TS
    is_left = xi < TS

    ri = jax.lax.broadcasted_iota(jnp.int32, (KA, K), 0)
    ci = jax.lax.broadcasted_iota(jnp.int32, (KA, K), 1)
    tri = (ci < ri).astype(_BF16)

    def body(k, carry):
        log_t, acc, ctl, ctr, cbl, cbr = carry
        pblk = params_ref[pl.ds(k * K, K), :]
        m0 = pblk[:, 0:1]
        m1 = pblk[:, 1:2]
        c00 = pblk[:, 2:3]
        c01 = pblk[:, 3:4]
        c10 = pblk[:, 4:5]
        c11 = pblk[:, 5:6]
        op = pblk[:, 6:7]

        det = c00 * c11 - c01 * c10
        inv_det = 1.0 / det
        pa = (-0.5 * _LOG2E) * (c11 * inv_det)
        pb = (0.5 * _LOG2E) * ((c01 + c10) * inv_det)
        pc = (-0.5 * _LOG2E) * (c00 * inv_det)
        pd = jnp.log2(op)

        mid = 0.5 * (c00 + c11)
        s = jnp.sqrt(jnp.maximum(mid * mid - det, 0.1))
        rad = 3.0 * jnp.ceil(jnp.sqrt(jnp.maximum(mid + s, mid - s)))

        rmin0 = jnp.clip(m0 - rad, 0.0, W - 1.0)
        rmax0 = jnp.clip(m0 + rad, 0.0, W - 1.0)
        rmin1 = jnp.clip(m1 - rad, 0.0, H - 1.0)
        rmax1 = jnp.clip(m1 + rad, 0.0, H - 1.0)
        mwl = (rmax0 >= 0.0) & (rmin0 <= TS - 1.0)
        mwr = (rmax0 >= float(TS)) & (rmin0 <= W - 1.0)
        mht = (rmax1 >= 0.0) & (rmin1 <= TS - 1.0)
        mhb = (rmax1 >= float(TS)) & (rmin1 <= H - 1.0)
        mtl = (mht & mwl).astype(_F32)
        mtr = (mht & mwr).astype(_F32)
        mbl = (mhb & mwl).astype(_F32)
        mbr = (mhb & mwr).astype(_F32)
        mask_f = jnp.where(is_top,
                           jnp.where(is_left, mtl, mtr),
                           jnp.where(is_left, mbl, mbr))

        u = y - m0
        v = x - m1
        e2 = u * (pa * u + pb * v) + pc * (v * v) + pd
        aval = jnp.exp2(e2) * mask_f
        lg = jnp.log2((1.0 - aval) + 1e-10)

        lg_hi = lg.astype(_BF16)
        lg_lo = (lg - lg_hi.astype(_F32)).astype(_BF16)
        cs = _dot_f32(tri, lg_hi) + _dot_f32(tri, lg_lo)
        tvals = jnp.exp2(cs[0:K, :] + log_t)
        bvals = tvals * aval

        cdt2 = cdt2_ref[:, pl.ds(k * K, K)]
        b_hi = bvals.astype(_BF16)
        b_lo = (bvals - b_hi.astype(_F32)).astype(_BF16)
        a2p = _dot_f32(cdt2, b_hi)
        acc = acc + a2p[0:8, :] + a2p[8:16, :] + _dot_f32(cdt2[0:8, :], b_lo)
        log_t = log_t + cs[K:K + 1, :]
        ctl = ctl + jnp.sum(mtl)
        ctr = ctr + jnp.sum(mtr)
        cbl = cbl + jnp.sum(mbl)
        cbr = cbr + jnp.sum(mbr)
        return log_t, acc, ctl, ctr, cbl, cbr

    log_t0 = jnp.zeros((1, P), _F32)
    acc0 = jnp.zeros((8, P), _F32)
    zero = jnp.float32(0.0)
    log_t, acc, ctl, ctr, cbl, cbr = jax.lax.fori_loop(
        0, NCHUNK, body, (log_t0, acc0, zero, zero, zero, zero))

    cnt = jnp.where(is_top,
                    jnp.where(is_left, ctl, ctr),
                    jnp.where(is_left, cbl, cbr))
    default = (jax.lax.broadcasted_iota(jnp.int32, (8, P), 0) < 3).astype(_F32)
    out_ref[...] = jnp.where(cnt > 0.0, acc, default)


def kernel(means2D, cov2d, color, opacity, depths):
    order = jnp.argsort(depths)
    tbl = jnp.concatenate(
        [means2D, cov2d.reshape(N, 4), opacity.reshape(N, 1),
         jnp.zeros((N, 1), _F32), color, depths.reshape(N, 1),
         jnp.ones((N, 1), _F32), jnp.zeros((N, 3), _F32)], axis=1)
    stbl = tbl[order]
    params = stbl[:, 0:8]
    cdt = stbl[:, 8:16].T
    cdt_hi = cdt.astype(_BF16)
    cdt_lo = (cdt - cdt_hi.astype(_F32)).astype(_BF16)
    cdt2 = jnp.concatenate([cdt_hi, cdt_lo], axis=0)
    covu = jnp.concatenate(
        [cov2d.reshape(N, 4).T, jnp.zeros((4, N), _F32)], axis=0)

    out, radii = pl.pallas_call(
        _blend_kernel,
        out_shape=[
            jax.ShapeDtypeStruct((8, P), _F32),
            jax.ShapeDtypeStruct((1, N), _F32),
        ],
    )(params, cdt2, covu)

    render_color = out[0:3].T.reshape(H, W, 3)
    render_depth = out[3].reshape(H, W, 1)
    render_alpha = out[4].reshape(H, W, 1)
    return render_color, render_depth, render_alpha, radii.reshape(N)


# speedup vs baseline: 25.2847x; 1.0363x over previous
import jax
import jax.numpy as jnp
from jax.experimental import pallas as pl

H = 64
W = 64
TS = 32
N = 4096
P = H * W
K = 256
KA = K + 8
NCHUNK = N // K

_F32 = jnp.float32
_BF16 = jnp.bfloat16
_HIGH = jax.lax.Precision.HIGHEST
_LOG2E = 1.4426950408889634


def _dot_f32(a, b):
    return jax.lax.dot(a, b, preferred_element_type=_F32)


def _blend_kernel(stbl_ref, covu_ref, out_ref, radii_ref):
    c00u = covu_ref[0:1, :]
    c01u = covu_ref[1:2, :]
    c10u = covu_ref[2:3, :]
    c11u = covu_ref[3:4, :]
    detu = c00u * c11u - c01u * c10u
    midu = 0.5 * (c00u + c11u)
    su = jnp.sqrt(jnp.maximum(midu * midu - detu, 0.1))
    radii_ref[...] = 3.0 * jnp.ceil(
        jnp.sqrt(jnp.maximum(midu + su, midu - su)))

    p_idx = jax.lax.broadcasted_iota(jnp.int32, (1, P), 1)
    yi = p_idx // W
    xi = p_idx - yi * W
    y = yi.astype(_F32)
    x = xi.astype(_F32)
    is_top = yi < TS
    is_left = xi < TS

    ri = jax.lax.broadcasted_iota(jnp.int32, (KA, K), 0)
    ci = jax.lax.broadcasted_iota(jnp.int32, (KA, K), 1)
    tri = (ci < ri).astype(_BF16)

    def body(k, carry):
        log_t, acc, ctl, ctr, cbl, cbr = carry
        pblk = stbl_ref[pl.ds(k * K, K), 0:8]
        m0 = pblk[:, 0:1]
        m1 = pblk[:, 1:2]
        c00 = pblk[:, 2:3]
        c01 = pblk[:, 3:4]
        c10 = pblk[:, 4:5]
        c11 = pblk[:, 5:6]
        op = pblk[:, 6:7]

        det = c00 * c11 - c01 * c10
        inv_det = 1.0 / det
        pa = (-0.5 * _LOG2E) * (c11 * inv_det)
        pb = (0.5 * _LOG2E) * ((c01 + c10) * inv_det)
        pc = (-0.5 * _LOG2E) * (c00 * inv_det)
        pd = jnp.log2(op)

        mid = 0.5 * (c00 + c11)
        s = jnp.sqrt(jnp.maximum(mid * mid - det, 0.1))
        rad = 3.0 * jnp.ceil(jnp.sqrt(jnp.maximum(mid + s, mid - s)))

        rmin0 = jnp.clip(m0 - rad, 0.0, W - 1.0)
        rmax0 = jnp.clip(m0 + rad, 0.0, W - 1.0)
        rmin1 = jnp.clip(m1 - rad, 0.0, H - 1.0)
        rmax1 = jnp.clip(m1 + rad, 0.0, H - 1.0)
        mwl = (rmax0 >= 0.0) & (rmin0 <= TS - 1.0)
        mwr = (rmax0 >= float(TS)) & (rmin0 <= W - 1.0)
        mht = (rmax1 >= 0.0) & (rmin1 <= TS - 1.0)
        mhb = (rmax1 >= float(TS)) & (rmin1 <= H - 1.0)
        mtl = (mht & mwl).astype(_F32)
        mtr = (mht & mwr).astype(_F32)
        mbl = (mhb & mwl).astype(_F32)
        mbr = (mhb & mwr).astype(_F32)
        mask_f = jnp.where(is_top,
                           jnp.where(is_left, mtl, mtr),
                           jnp.where(is_left, mbl, mbr))

        u = y - m0
        v = x - m1
        e2 = u * (pa * u + pb * v) + pc * (v * v) + pd
        aval = jnp.exp2(e2) * mask_f
        lg = jnp.log2((1.0 - aval) + 1e-10)

        lg_hi = lg.astype(_BF16)
        lg_lo = (lg - lg_hi.astype(_F32)).astype(_BF16)
        cs = _dot_f32(tri, lg_hi) + _dot_f32(tri, lg_lo)
        tvals = jnp.exp2(cs[0:K, :] + log_t)
        bvals = tvals * aval

        cdt = stbl_ref[pl.ds(k * K, K), 8:16].T
        cdt_hi = cdt.astype(_BF16)
        cdt_lo = (cdt - cdt_hi.astype(_F32)).astype(_BF16)
        cdt2 = jnp.concatenate([cdt_hi, cdt_lo], axis=0)
        b_hi = bvals.astype(_BF16)
        b_lo = (bvals - b_hi.astype(_F32)).astype(_BF16)
        a2p = _dot_f32(cdt2, b_hi)
        acc = acc + a2p[0:8, :] + a2p[8:16, :] + _dot_f32(cdt_hi, b_lo)
        log_t = log_t + cs[K:K + 1, :]
        ctl = ctl + jnp.sum(mtl)
        ctr = ctr + jnp.sum(mtr)
        cbl = cbl + jnp.sum(mbl)
        cbr = cbr + jnp.sum(mbr)
        return log_t, acc, ctl, ctr, cbl, cbr

    log_t0 = jnp.zeros((1, P), _F32)
    acc0 = jnp.zeros((8, P), _F32)
    zero = jnp.float32(0.0)
    log_t, acc, ctl, ctr, cbl, cbr = jax.lax.fori_loop(
        0, NCHUNK, body, (log_t0, acc0, zero, zero, zero, zero))

    cnt = jnp.where(is_top,
                    jnp.where(is_left, ctl, ctr),
                    jnp.where(is_left, cbl, cbr))
    default = (jax.lax.broadcasted_iota(jnp.int32, (8, P), 0) < 3).astype(_F32)
    out_ref[...] = jnp.where(cnt > 0.0, acc, default)


def kernel(means2D, cov2d, color, opacity, depths):
    order = jnp.argsort(depths)
    tbl = jnp.concatenate(
        [means2D, cov2d.reshape(N, 4), opacity.reshape(N, 1),
         jnp.zeros((N, 1), _F32), color, depths.reshape(N, 1),
         jnp.ones((N, 1), _F32), jnp.zeros((N, 3), _F32)], axis=1)
    stbl = tbl[order]
    covu = cov2d.reshape(N, 4).T

    out, radii = pl.pallas_call(
        _blend_kernel,
        out_shape=[
            jax.ShapeDtypeStruct((8, P), _F32),
            jax.ShapeDtypeStruct((1, N), _F32),
        ],
    )(stbl, covu)

    render_color = out[0:3].T.reshape(H, W, 3)
    render_depth = out[3].reshape(H, W, 1)
    render_alpha = out[4].reshape(H, W, 1)
    return render_color, render_depth, render_alpha, radii.reshape(N)
